# baseline (speedup 1.0000x reference)
import functools
import math
import random

import jax
import jax.numpy as jnp
import numpy as np
from jax import lax
from jax.experimental import pallas as pl
from jax.experimental.pallas import tpu as pltpu
from jax.experimental.pallas import tpu_sc as plsc

_NW = 32
_CHUNK = 256
_NBUF = 2


def _algorithm_l_map(R, N):
    rng = random.Random(0)
    eps = 1e-06

    def u():
        return min(max(rng.random(), eps), 1.0 - eps)

    w_gen = 1.0
    current_index = R + 1
    cmap = {}
    while current_index <= N:
        candidate_idx = current_index - 1
        updated_idx = rng.randrange(R)
        cmap[updated_idx] = candidate_idx
        w_gen *= math.exp(math.log(u()) / R)
        w_gen = min(max(w_gen, 1e-06), 1.0 - 1e-06)
        current_index += math.floor(math.log(u()) / math.log(1.0 - w_gen)) + 1
    return cmap


@functools.lru_cache(maxsize=None)
def _patch_tables(R, N, D):
    cols_per_w = R // _NW
    nchunk = cols_per_w // _CHUNK
    groups = [[[] for _ in range(nchunk)] for _ in range(_NW)]
    for slot, cand in _algorithm_l_map(R, N).items():
        w, r = divmod(slot, cols_per_w)
        c, pos = divmod(r, _CHUNK)
        groups[w][c].append((cand // 2, (pos << 8) | (64 * (cand % 2))))
    maxfix = max(len(g) for row in groups for g in row)
    maxfix = (maxfix + 15) & ~15
    ridx = np.empty((_NW, nchunk, maxfix), dtype=np.int32)
    ptab = np.empty((_NW, nchunk, maxfix), dtype=np.int32)
    for w in range(_NW):
        for c in range(nchunk):
            g = groups[w][c]
            g = g + [g[0]] * (maxfix - len(g))
            ridx[w, c, :] = [e[0] for e in g]
            ptab[w, c, :] = [e[1] for e in g]
    return ridx.reshape(-1), ptab.reshape(-1), maxfix, nchunk


@functools.lru_cache(maxsize=None)
def _sampler_kernel(N, R, D, maxfix, nchunk):
    cols_per_w = R // _NW
    tab_per_w = nchunk * maxfix
    ngroups = maxfix // 16
    mesh = plsc.VectorSubcoreMesh(core_axis_name="c", subcore_axis_name="s")

    @functools.partial(
        pl.kernel,
        mesh=mesh,
        compiler_params=pltpu.CompilerParams(
            use_tc_tiling_on_sc=True, needs_layout_passes=False
        ),
        out_type=jax.ShapeDtypeStruct((D, R), jnp.float32),
        scratch_types=[
            pltpu.VMEM((tab_per_w,), jnp.int32),
            pltpu.VMEM((tab_per_w,), jnp.int32),
            pltpu.VMEM((_NBUF, D, _CHUNK), jnp.float32),
            pltpu.VMEM((_NBUF, maxfix, 2 * D), jnp.float32),
            pltpu.SemaphoreType.DMA((_NBUF,)),
            pltpu.SemaphoreType.DMA((_NBUF,)),
            pltpu.SemaphoreType.DMA((_NBUF,)),
        ],
    )
    def k(xt_hbm, xp_hbm, ridx_hbm, ptab_hbm, ot_hbm, ridx_v, ptab_v, buf_v,
          stag_v, bsem, psem, wsem):
        lane = lax.iota(jnp.int32, 16)
        wid = lax.axis_index("s") * 2 + lax.axis_index("c")
        base = wid * cols_per_w
        toff = wid * tab_per_w
        pltpu.sync_copy(ridx_hbm.at[pl.ds(toff, tab_per_w)], ridx_v)
        pltpu.sync_copy(ptab_hbm.at[pl.ds(toff, tab_per_w)], ptab_v)

        def issue(j):
            b = j % _NBUF
            pltpu.async_copy(
                xt_hbm.at[:, pl.ds(base + j * _CHUNK, _CHUNK)], buf_v.at[b],
                bsem.at[b],
            )
            pltpu.async_copy(
                xp_hbm.at[ridx_v.at[pl.ds(j * maxfix, maxfix)]], stag_v.at[b],
                psem.at[b],
            )

        def apply_patches(j):
            b = j % _NBUF

            def group(g, _):
                vec = ptab_v[pl.ds(j * maxfix + g * 16, 16)]
                for t in range(16):
                    v = vec[t]
                    pos = lax.shift_right_logical(v, 8)
                    soff = lax.bitwise_and(v, 0xFF)
                    col = jnp.full((16,), pos, jnp.int32)
                    for q in range(4):
                        val = stag_v[b, g * 16 + t, pl.ds(soff + q * 16, 16)]
                        plsc.store_scatter(
                            buf_v.at[b], [lane + q * 16, col], val
                        )
                return 0

            lax.fori_loop(0, ngroups, group, 0)

        def wait_in(j):
            b = j % _NBUF
            pltpu.make_async_copy(
                xt_hbm.at[:, pl.ds(base + j * _CHUNK, _CHUNK)], buf_v.at[b],
                bsem.at[b],
            ).wait()
            pltpu.make_async_copy(
                xp_hbm.at[pl.ds(0, maxfix)], stag_v.at[b], psem.at[b]
            ).wait()

        def writeback(j):
            b = j % _NBUF
            pltpu.async_copy(
                buf_v.at[b], ot_hbm.at[:, pl.ds(base + j * _CHUNK, _CHUNK)],
                wsem.at[b],
            )

        def wait_wb(j):
            b = j % _NBUF
            pltpu.make_async_copy(
                buf_v.at[b], ot_hbm.at[:, pl.ds(base + j * _CHUNK, _CHUNK)],
                wsem.at[b],
            ).wait()

        issue(0)
        for j in range(nchunk):
            nj = j + 1
            if nj < nchunk:
                if nj >= _NBUF:
                    wait_wb(nj - _NBUF)
                issue(nj)
            wait_in(j)
            apply_patches(j)
            writeback(j)
        for j in range(max(0, nchunk - _NBUF), nchunk):
            wait_wb(j)

    return k


def kernel(x_nd, r_ld):
    R, D = r_ld.shape
    N = x_nd.shape[0]
    ridx, ptab, maxfix, nchunk = _patch_tables(R, N, D)
    xp = jnp.reshape(x_nd, (N // 2, 2 * D))
    ot = _sampler_kernel(N, R, D, maxfix, nchunk)(
        x_nd.T, xp, jnp.asarray(ridx), jnp.asarray(ptab)
    )
    return ot.T


# speedup vs baseline: 1.0579x; 1.0579x over previous
import functools
import math
import random

import jax
import jax.numpy as jnp
import numpy as np
from jax import lax
from jax.experimental import pallas as pl
from jax.experimental.pallas import tpu as pltpu
from jax.experimental.pallas import tpu_sc as plsc

_NW = 32
_TCCH = 4
_NBUF = 2


def _algorithm_l_map(R, N):
    rng = random.Random(0)
    eps = 1e-06

    def u():
        return min(max(rng.random(), eps), 1.0 - eps)

    w_gen = 1.0
    current_index = R + 1
    cmap = {}
    while current_index <= N:
        candidate_idx = current_index - 1
        updated_idx = rng.randrange(R)
        cmap[updated_idx] = candidate_idx
        w_gen *= math.exp(math.log(u()) / R)
        w_gen = min(max(w_gen, 1e-06), 1.0 - 1e-06)
        current_index += math.floor(math.log(u()) / math.log(1.0 - w_gen)) + 1
    return cmap


@functools.lru_cache(maxsize=None)
def _patch_tables(R, N):
    rows_per_w = R // _NW
    chunk_rows = _TCCH * 128
    nchunk = rows_per_w // chunk_rows
    groups = [[[] for _ in range(nchunk)] for _ in range(_NW)]
    for slot, cand in _algorithm_l_map(R, N).items():
        w, r = divmod(slot, rows_per_w)
        c, i_rel = divmod(r, chunk_rows)
        groups[w][c].append((cand, (i_rel // 128) << 7 | (i_rel % 128)))
    maxfix = max(len(g) for row in groups for g in row)
    maxfix = (maxfix + 15) & ~15
    ridx = np.empty((_NW, nchunk, maxfix), dtype=np.int32)
    ptab = np.empty((_NW, nchunk, maxfix), dtype=np.int32)
    for w in range(_NW):
        for c in range(nchunk):
            g = groups[w][c]
            g = g + [g[0]] * (maxfix - len(g))
            ridx[w, c, :] = [e[0] for e in g]
            ptab[w, c, :] = [e[1] for e in g]
    return ridx.reshape(-1), ptab.reshape(-1), maxfix, nchunk


@functools.lru_cache(maxsize=None)
def _sampler_kernel(N, R, D, maxfix, nchunk):
    NTJ = D // 8
    ti_per_w = (R // 128) // _NW
    tab_per_w = nchunk * maxfix
    ngroups = maxfix // 16
    inner_ch = _TCCH * 1024
    mesh = plsc.VectorSubcoreMesh(core_axis_name="c", subcore_axis_name="s")

    @functools.partial(
        pl.kernel,
        mesh=mesh,
        compiler_params=pltpu.CompilerParams(
            use_tc_tiling_on_sc=False, needs_layout_passes=False
        ),
        out_type=jax.ShapeDtypeStruct((NTJ, R // 128, 8, 128), jnp.float32),
        scratch_types=[
            pltpu.VMEM((tab_per_w,), jnp.int32),
            pltpu.VMEM((tab_per_w,), jnp.int32),
            pltpu.VMEM((_NBUF, NTJ, _TCCH, 8, 128), jnp.float32),
            pltpu.VMEM((_NBUF, maxfix, D), jnp.float32),
            pltpu.SemaphoreType.DMA((_NBUF,)),
            pltpu.SemaphoreType.DMA((_NBUF,)),
            pltpu.SemaphoreType.DMA((_NBUF,)),
        ],
    )
    def k(xv_hbm, xr_hbm, ridx_hbm, ptab_hbm, ov_hbm, ridx_v, ptab_v, buf_v,
          stag_v, bsem, psem, wsem):
        lane = lax.iota(jnp.int32, 16)
        wid = lax.axis_index("s") * 2 + lax.axis_index("c")
        base = wid * ti_per_w
        toff = wid * tab_per_w
        pltpu.sync_copy(ridx_hbm.at[pl.ds(toff, tab_per_w)], ridx_v)
        pltpu.sync_copy(ptab_hbm.at[pl.ds(toff, tab_per_w)], ptab_v)

        tjv = []
        jjv = []
        for q in range(4):
            idx64 = lane + q * 16
            tjv.append(lax.shift_right_logical(idx64, 3))
            jjv.append(lax.bitwise_and(idx64, 7))

        def issue(j):
            b = j % _NBUF
            for tj in range(NTJ):
                pltpu.async_copy(
                    xv_hbm.at[tj, pl.ds(base + j * _TCCH, _TCCH)],
                    buf_v.at[b, tj],
                    bsem.at[b],
                )
            pltpu.async_copy(
                xr_hbm.at[ridx_v.at[pl.ds(j * maxfix, maxfix)]], stag_v.at[b],
                psem.at[b],
            )

        def wait_in(j):
            b = j % _NBUF
            pltpu.make_async_copy(
                xv_hbm.at[:, pl.ds(0, _TCCH)], buf_v.at[b], bsem.at[b]
            ).wait()
            pltpu.make_async_copy(
                xr_hbm.at[pl.ds(0, maxfix)], stag_v.at[b], psem.at[b]
            ).wait()

        def apply_patches(j):
            b = j % _NBUF

            def group(g, _):
                vec = ptab_v[pl.ds(j * maxfix + g * 16, 16)]
                for t in range(16):
                    e = vec[t]
                    crel = jnp.full((16,), lax.shift_right_logical(e, 7),
                                    jnp.int32)
                    dv = jnp.full((16,), lax.bitwise_and(e, 127), jnp.int32)
                    for q in range(4):
                        val = stag_v[b, g * 16 + t, pl.ds(q * 16, 16)]
                        plsc.store_scatter(
                            buf_v.at[b],
                            [tjv[q], crel, jjv[q], dv],
                            val,
                        )
                return 0

            lax.fori_loop(0, ngroups, group, 0)

        def writeback(j):
            b = j % _NBUF
            for tj in range(NTJ):
                pltpu.async_copy(
                    buf_v.at[b, tj],
                    ov_hbm.at[tj, pl.ds(base + j * _TCCH, _TCCH)],
                    wsem.at[b],
                )

        def wait_wb(j):
            b = j % _NBUF
            pltpu.make_async_copy(
                buf_v.at[b], ov_hbm.at[:, pl.ds(0, _TCCH)], wsem.at[b]
            ).wait()

        issue(0)
        for j in range(nchunk):
            nj = j + 1
            if nj < nchunk:
                if nj >= _NBUF:
                    wait_wb(nj - _NBUF)
                issue(nj)
            wait_in(j)
            apply_patches(j)
            writeback(j)
        for j in range(max(0, nchunk - _NBUF), nchunk):
            wait_wb(j)

    return k


def kernel(x_nd, r_ld):
    R, D = r_ld.shape
    N = x_nd.shape[0]
    ridx, ptab, maxfix, nchunk = _patch_tables(R, N)
    xv = x_nd.T.reshape(D // 8, 8, N // 128, 128).transpose(0, 2, 1, 3)
    xr = x_nd
    ov = _sampler_kernel(N, R, D, maxfix, nchunk)(
        xv, xr, jnp.asarray(ridx), jnp.asarray(ptab)
    )
    out = ov.transpose(0, 2, 1, 3).reshape(D, R).T
    return out


# speedup vs baseline: 1.0652x; 1.0069x over previous
import functools
import math
import random

import jax
import jax.numpy as jnp
import numpy as np
from jax import lax
from jax.experimental import pallas as pl
from jax.experimental.pallas import tpu as pltpu
from jax.experimental.pallas import tpu_sc as plsc

_NW = 32
_TCCH = 4
_NBUF = 2


def _algorithm_l_map(R, N):
    rng = random.Random(0)
    eps = 1e-06

    def u():
        return min(max(rng.random(), eps), 1.0 - eps)

    w_gen = 1.0
    current_index = R + 1
    cmap = {}
    while current_index <= N:
        candidate_idx = current_index - 1
        updated_idx = rng.randrange(R)
        cmap[updated_idx] = candidate_idx
        w_gen *= math.exp(math.log(u()) / R)
        w_gen = min(max(w_gen, 1e-06), 1.0 - 1e-06)
        current_index += math.floor(math.log(u()) / math.log(1.0 - w_gen)) + 1
    return cmap


@functools.lru_cache(maxsize=None)
def _patch_tables(R, N):
    rows_per_w = R // _NW
    chunk_rows = _TCCH * 128
    nchunk = rows_per_w // chunk_rows
    groups = [[[] for _ in range(nchunk)] for _ in range(_NW)]
    for slot, cand in _algorithm_l_map(R, N).items():
        w, r = divmod(slot, rows_per_w)
        c, i_rel = divmod(r, chunk_rows)
        groups[w][c].append((cand, (i_rel // 128) * 1024 + (i_rel % 128)))
    maxfix = max(len(g) for row in groups for g in row)
    maxfix = (maxfix + 15) & ~15
    ridx = np.empty((_NW, nchunk, maxfix), dtype=np.int32)
    ptab = np.empty((_NW, nchunk, maxfix), dtype=np.int32)
    for w in range(_NW):
        for c in range(nchunk):
            g = groups[w][c]
            g = g + [g[0]] * (maxfix - len(g))
            ridx[w, c, :] = [e[0] for e in g]
            ptab[w, c, :] = [e[1] for e in g]
    return ridx.reshape(-1), ptab.reshape(-1), maxfix, nchunk


@functools.lru_cache(maxsize=None)
def _sampler_kernel(N, R, D, maxfix, nchunk):
    NTJ = D // 8
    ti_per_w = (R // 128) // _NW
    tab_per_w = nchunk * maxfix
    ngroups = maxfix // 16
    inner_ch = _TCCH * 1024
    mesh = plsc.VectorSubcoreMesh(core_axis_name="c", subcore_axis_name="s")

    @functools.partial(
        pl.kernel,
        mesh=mesh,
        compiler_params=pltpu.CompilerParams(
            use_tc_tiling_on_sc=False, needs_layout_passes=False,
            disable_bounds_checks=True,
        ),
        out_type=jax.ShapeDtypeStruct((NTJ, R // 128, 8, 128), jnp.float32),
        scratch_types=[
            pltpu.VMEM((tab_per_w,), jnp.int32),
            pltpu.VMEM((tab_per_w,), jnp.int32),
            pltpu.VMEM((_NBUF, NTJ, _TCCH, 8, 128), jnp.float32),
            pltpu.VMEM((_NBUF, maxfix, D), jnp.float32),
            pltpu.SemaphoreType.DMA((_NBUF,)),
            pltpu.SemaphoreType.DMA((_NBUF,)),
            pltpu.SemaphoreType.DMA((_NBUF,)),
        ],
    )
    def k(xv_hbm, xr_hbm, ridx_hbm, ptab_hbm, ov_hbm, ridx_v, ptab_v, buf_v,
          stag_v, bsem, psem, wsem):
        lane = lax.iota(jnp.int32, 16)
        wid = lax.axis_index("s") * 2 + lax.axis_index("c")
        base = wid * ti_per_w
        toff = wid * tab_per_w
        pltpu.sync_copy(ridx_hbm.at[pl.ds(toff, tab_per_w)], ridx_v)
        pltpu.sync_copy(ptab_hbm.at[pl.ds(toff, tab_per_w)], ptab_v)

        tjv = []
        jjv = []
        zero = jnp.full((16,), 0, jnp.int32)
        for q in range(4):
            idx64 = lane + q * 16
            tjv.append(lax.shift_right_logical(idx64, 3))
            jjv.append(lax.bitwise_and(idx64, 7))

        def issue(j):
            b = j % _NBUF
            for tj in range(NTJ):
                pltpu.async_copy(
                    xv_hbm.at[tj, pl.ds(base + j * _TCCH, _TCCH)],
                    buf_v.at[b, tj],
                    bsem.at[b],
                )
            pltpu.async_copy(
                xr_hbm.at[ridx_v.at[pl.ds(j * maxfix, maxfix)]], stag_v.at[b],
                psem.at[b],
            )

        def wait_in(j):
            b = j % _NBUF
            pltpu.make_async_copy(
                xv_hbm.at[:, pl.ds(0, _TCCH)], buf_v.at[b], bsem.at[b]
            ).wait()
            pltpu.make_async_copy(
                xr_hbm.at[pl.ds(0, maxfix)], stag_v.at[b], psem.at[b]
            ).wait()

        def apply_patches(j):
            b = j % _NBUF

            def group(g, _):
                vec = ptab_v[pl.ds(j * maxfix + g * 16, 16)]
                for t in range(16):
                    ev = jnp.full((16,), vec[t], jnp.int32)
                    for q in range(4):
                        val = stag_v[b, g * 16 + t, pl.ds(q * 16, 16)]
                        plsc.store_scatter(
                            buf_v.at[b],
                            [tjv[q], zero, jjv[q], ev],
                            val,
                        )
                return 0

            lax.fori_loop(0, ngroups, group, 0)

        def writeback(j):
            b = j % _NBUF
            for tj in range(NTJ):
                pltpu.async_copy(
                    buf_v.at[b, tj],
                    ov_hbm.at[tj, pl.ds(base + j * _TCCH, _TCCH)],
                    wsem.at[b],
                )

        def wait_wb(j):
            b = j % _NBUF
            pltpu.make_async_copy(
                buf_v.at[b], ov_hbm.at[:, pl.ds(0, _TCCH)], wsem.at[b]
            ).wait()

        issue(0)
        for j in range(nchunk):
            nj = j + 1
            if nj < nchunk:
                if nj >= _NBUF:
                    wait_wb(nj - _NBUF)
                issue(nj)
            wait_in(j)
            apply_patches(j)
            writeback(j)
        for j in range(max(0, nchunk - _NBUF), nchunk):
            wait_wb(j)

    return k


def kernel(x_nd, r_ld):
    R, D = r_ld.shape
    N = x_nd.shape[0]
    ridx, ptab, maxfix, nchunk = _patch_tables(R, N)
    xv = x_nd.T.reshape(D // 8, 8, N // 128, 128).transpose(0, 2, 1, 3)
    xr = x_nd
    ov = _sampler_kernel(N, R, D, maxfix, nchunk)(
        xv, xr, jnp.asarray(ridx), jnp.asarray(ptab)
    )
    out = ov.transpose(0, 2, 1, 3).reshape(D, R).T
    return out
